# baseline (speedup 1.0000x reference)
import functools

import jax
import jax.numpy as jnp
from jax import lax
from jax.experimental import pallas as pl
from jax.experimental.pallas import tpu as pltpu
from jax.experimental.pallas import tpu_sc as plsc

QUES = 2048
SEQH = 2048
T = 8192
K = 32
KC = 40
NC2 = 2 * KC

G1 = 16
WROWS = (3 * SEQH) // G1

NSUB = 16
RPS = T // (2 * NSUB)
CHU = 16
NCH = RPS // CHU
NEG = -3.0e38


def _matvec_body(sel_ref, q_ref, h_ref, wih_ref, whh_ref, gi_ref, gh_ref):
    q = q_ref[...]
    h = h_ref[...]
    gi_ref[...] = jnp.dot(wih_ref[...], q,
                          preferred_element_type=jnp.float32)
    gh_ref[...] = jnp.dot(whh_ref[...], h,
                          preferred_element_type=jnp.float32)


def _sc_alpha_body(ques, q, idx_out, qc_out,
                   hv, wch, av, lvv, liv, cvv, civ, rows,
                   sv_sh, si_sh, sem):
    cid = lax.axis_index("c")
    sid = lax.axis_index("s")
    lane = lax.broadcasted_iota(jnp.int32, (16,), 0)
    m0 = lane == 0
    zf = jnp.zeros((16,), jnp.float32)
    zi = jnp.zeros((16,), jnp.int32)

    wid = cid * NSUB + sid
    base = pl.multiple_of(wid * RPS, RPS)
    pltpu.sync_copy(q, hv)

    for ch in range(NCH):
        row0 = base + ch * CHU
        pltpu.sync_copy(ques.at[pl.ds(row0, CHU)], wch)
        for rb in range(CHU // 8):
            def cstep(c, accs, rb=rb):
                hc = hv[pl.ds(c * 16, 16)]
                return tuple(
                    accs[r] + wch.at[rb * 8 + r][pl.ds(c * 16, 16)] * hc
                    for r in range(8))

            accs = lax.fori_loop(0, QUES // 16, cstep, (zf,) * 8)
            for r in range(8):
                pos = ch * CHU + rb * 8 + r
                plsc.store_scatter(av, [zi + pos],
                                   zf + jnp.sum(accs[r]), mask=m0)

    def fold(t, carry):
        vb, ib = carry
        v = av[pl.ds(t * 16, 16)]
        gidx = base + t * 16 + lane
        better = v > vb
        return jnp.where(better, v, vb), jnp.where(better, gidx, ib)

    def ex(p, _):
        vb, ib = lax.fori_loop(1, RPS // 16, fold,
                               (av[pl.ds(0, 16)], base + lane))
        mval = jnp.max(vb)
        midx = jnp.max(jnp.where(vb == mval, ib, jnp.int32(-1)))
        pv = zi + p
        plsc.store_scatter(lvv, [pv], zf + mval, mask=m0)
        plsc.store_scatter(liv, [pv], zi + midx, mask=m0)
        plsc.store_scatter(av, [zi + (midx - base)], zf + NEG, mask=m0)
        return 0

    lax.fori_loop(0, KC, ex, 0)
    pltpu.sync_copy(lvv, sv_sh.at[pl.ds(sid * KC, KC)])
    pltpu.sync_copy(liv, si_sh.at[pl.ds(sid * KC, KC)])
    plsc.subcore_barrier()

    @pl.when(sid == 0)
    def _():
        pltpu.sync_copy(sv_sh, cvv)
        pltpu.sync_copy(si_sh, civ)

        def gfold(t, carry):
            vb, ib = carry
            v = cvv[pl.ds(t * 16, 16)]
            slot = t * 16 + lane
            better = v > vb
            return jnp.where(better, v, vb), jnp.where(better, slot, ib)

        def gex(p, _):
            vb, ib = lax.fori_loop(1, NSUB * KC // 16, gfold,
                                   (cvv[pl.ds(0, 16)], lane))
            mval = jnp.max(vb)
            mslot = jnp.max(jnp.where(vb == mval, ib, jnp.int32(-1)))
            slotv = zi + mslot
            orig = plsc.load_gather(civ, [slotv])
            pv = zi + p
            plsc.store_scatter(liv, [pv], orig, mask=m0)
            plsc.store_scatter(cvv, [slotv], zf + NEG, mask=m0)
            return 0

        lax.fori_loop(0, KC, gex, 0)
        off = pl.multiple_of(cid * KC, KC)
        pltpu.sync_copy(liv, idx_out.at[pl.ds(off, KC)])
        pltpu.async_copy(ques.at[liv], rows, sem).wait()
        pltpu.sync_copy(rows, qc_out.at[pl.ds(off, KC)])


def _sc_alpha(questions, question):
    f32 = jnp.float32
    i32 = jnp.int32
    mesh = plsc.VectorSubcoreMesh(core_axis_name="c", subcore_axis_name="s")
    return pl.kernel(
        _sc_alpha_body,
        mesh=mesh,
        compiler_params=pltpu.CompilerParams(needs_layout_passes=False,
                                             use_tc_tiling_on_sc=False),
        out_type=[
            jax.ShapeDtypeStruct((NC2,), i32),
            jax.ShapeDtypeStruct((NC2, QUES), f32),
        ],
        scratch_types=[
            pltpu.VMEM((QUES,), f32),
            pltpu.VMEM((CHU, QUES), f32),
            pltpu.VMEM((RPS,), f32),
            pltpu.VMEM((KC,), f32),
            pltpu.VMEM((KC,), i32),
            pltpu.VMEM((NSUB * KC,), f32),
            pltpu.VMEM((NSUB * KC,), i32),
            pltpu.VMEM((KC, QUES), f32),
            pltpu.VMEM_SHARED((NSUB * KC,), f32),
            pltpu.VMEM_SHARED((NSUB * KC,), i32),
            pltpu.SemaphoreType.DMA,
        ],
    )(questions, question)


def _final_body(idx_ref, qc_ref, row_ref, q_ref, ws_ref, bs_ref,
                gi_ref, gh_ref, h_ref, bih_ref, bhh_ref,
                pred_ref, hnew_ref, acc_ref, w_ref):
    i = pl.program_id(0)
    iota = lax.broadcasted_iota(jnp.int32, (NC2, 1), 0)

    @pl.when(i == 0)
    def _():
        vals = jnp.dot(qc_ref[...], q_ref[...],
                       preferred_element_type=jnp.float32)
        neg = jnp.float32(-jnp.inf)

        def step(j, carry):
            a, win = carry
            m = jnp.max(a)
            pos = jnp.min(jnp.where(a == m, iota, NC2))
            win = jnp.where(iota == pos, 1.0, win)
            a = jnp.where(iota == pos, neg, a)
            return a, win

        _, win = lax.fori_loop(0, K, step,
                               (vals, jnp.zeros((NC2, 1), jnp.float32)))
        mx = jnp.max(jnp.where(win > 0, vals, neg))
        e = jnp.where(win > 0, jnp.exp(vals - mx), 0.0)
        w_ref[...] = e / jnp.sum(e)
        acc_ref[...] = jnp.zeros_like(acc_ref)

    wi = jnp.sum(jnp.where(iota == i, w_ref[...], 0.0))
    acc_ref[...] += wi * row_ref[0]

    @pl.when(i == NC2 - 1)
    def _():
        ws = ws_ref[...]
        qrow = q_ref[...].reshape(1, QUES)
        pred = (jnp.sum(ws[0:1] * qrow)
                + jnp.sum(ws[1:2] * acc_ref[...]) + bs_ref[0, 0])
        pred_ref[...] = pred[None, None]
        gi = gi_ref[...] + bih_ref[...]
        gh = gh_ref[...] + bhh_ref[...]
        h = h_ref[...]
        r = jax.nn.sigmoid(gi[0:16] + gh[0:16])
        z = jax.nn.sigmoid(gi[16:32] + gh[16:32])
        n = jnp.tanh(gi[32:48] + r * gh[32:48])
        hnew_ref[...] = (1.0 - z) * n + z * h


def kernel(question, score, questions, hs, Ws, bs, W_ih, W_hh, b_ih, b_hh):
    f32 = jnp.float32
    q2 = question.reshape(QUES, 1)
    h_prev = hs[T - 1, 0]
    h2 = h_prev.reshape(SEQH, 1)
    sel = (score[0] < 0.5).astype(jnp.int32).reshape(1)

    cand_i, qcand = _sc_alpha(questions, question)

    grid_spec = pltpu.PrefetchScalarGridSpec(
        num_scalar_prefetch=1,
        grid=(G1,),
        in_specs=[
            pl.BlockSpec((QUES, 1), lambda i, s: (0, 0)),
            pl.BlockSpec((SEQH, 1), lambda i, s: (0, 0)),
            pl.BlockSpec((WROWS, QUES), lambda i, s: (i, s[0])),
            pl.BlockSpec((WROWS, SEQH), lambda i, s: (i, 0)),
        ],
        out_specs=[
            pl.BlockSpec((WROWS, 1), lambda i, s: (i, 0)),
            pl.BlockSpec((WROWS, 1), lambda i, s: (i, 0)),
        ],
    )
    gi, gh = pl.pallas_call(
        _matvec_body,
        grid_spec=grid_spec,
        out_shape=[
            jax.ShapeDtypeStruct((3 * SEQH, 1), f32),
            jax.ShapeDtypeStruct((3 * SEQH, 1), f32),
        ],
    )(sel, q2, h2, W_ih, W_hh)

    pred, h_new = pl.pallas_call(
        _final_body,
        grid_spec=pltpu.PrefetchScalarGridSpec(
            num_scalar_prefetch=1,
            grid=(NC2,),
            in_specs=[
                pl.BlockSpec((NC2, QUES), lambda i, s: (0, 0)),
                pl.BlockSpec((1, 1, SEQH), lambda i, s: (s[i], 0, 0)),
                pl.BlockSpec((QUES, 1), lambda i, s: (0, 0)),
                pl.BlockSpec((2, QUES), lambda i, s: (0, 0)),
                pl.BlockSpec((1, 1), lambda i, s: (0, 0)),
                pl.BlockSpec((48, 128), lambda i, s: (0, 0)),
                pl.BlockSpec((48, 128), lambda i, s: (0, 0)),
                pl.BlockSpec((16, 128), lambda i, s: (0, 0)),
                pl.BlockSpec((48, 128), lambda i, s: (0, 0)),
                pl.BlockSpec((48, 128), lambda i, s: (0, 0)),
            ],
            out_specs=[
                pl.BlockSpec((1, 1), lambda i, s: (0, 0)),
                pl.BlockSpec((16, 128), lambda i, s: (0, 0)),
            ],
            scratch_shapes=[
                pltpu.VMEM((1, SEQH), f32),
                pltpu.VMEM((NC2, 1), f32),
            ],
        ),
        out_shape=[
            jax.ShapeDtypeStruct((1, 1), f32),
            jax.ShapeDtypeStruct((16, 128), f32),
        ],
    )(
        cand_i, qcand, hs,
        q2, Ws.reshape(2, QUES), bs.reshape(1, 1),
        gi.reshape(48, 128), gh.reshape(48, 128), h_prev.reshape(16, 128),
        b_ih.reshape(48, 128), b_hh.reshape(48, 128),
    )
    return (pred.reshape(1), h_new.reshape(1, 1, SEQH))


# speedup vs baseline: 1.8613x; 1.8613x over previous
import functools

import jax
import jax.numpy as jnp
from jax import lax
from jax.experimental import pallas as pl
from jax.experimental.pallas import tpu as pltpu

QUES = 2048
SEQH = 2048
T = 8192
K = 32

G1 = 32
QROWS = T // G1
WROWS = (3 * SEQH) // G1


def _matvec_body(sel_ref, q_ref, h_ref, ques_ref, wih_ref, whh_ref,
                 alpha_ref, gi_ref, gh_ref):
    q = q_ref[...]
    h = h_ref[...]
    alpha_ref[...] = jnp.dot(ques_ref[...], q,
                             preferred_element_type=jnp.float32)
    gi_ref[...] = jnp.dot(wih_ref[...], q,
                          preferred_element_type=jnp.float32)
    gh_ref[...] = jnp.dot(whh_ref[...], h,
                          preferred_element_type=jnp.float32)


def _topk_body(alpha_ref, idx_ref, w_ref):
    a = alpha_ref[...]
    iota = (lax.broadcasted_iota(jnp.int32, (64, 128), 0) * 128
            + lax.broadcasted_iota(jnp.int32, (64, 128), 1))
    kiota = lax.broadcasted_iota(jnp.int32, (1, K), 1)
    neg = jnp.float32(-jnp.inf)

    def step(j, carry):
        a, idxs, vals = carry
        m = jnp.max(a)
        idx = jnp.min(jnp.where(a == m, iota, T))
        idxs = jnp.where(kiota == j, idx, idxs)
        vals = jnp.where(kiota == j, m, vals)
        a = jnp.where(iota == idx, neg, a)
        return a, idxs, vals

    idxs0 = jnp.zeros((1, K), jnp.int32)
    vals0 = jnp.full((1, K), neg, jnp.float32)
    _, idxs, vals = lax.fori_loop(0, K, step, (a, idxs0, vals0))
    e = jnp.exp(vals - jnp.max(vals))
    w = e / jnp.sum(e)
    idx_ref[...] = idxs
    w_ref[...] = w


def _final_body(idx_ref, w_ref, row_ref, q_ref, ws_ref, bs_ref,
                gi_ref, gh_ref, h_ref, bih_ref, bhh_ref,
                pred_ref, hnew_ref, acc_ref):
    i = pl.program_id(0)

    @pl.when(i == 0)
    def _():
        acc_ref[...] = jnp.zeros_like(acc_ref)

    kiota = lax.broadcasted_iota(jnp.int32, (1, K), 1)
    wi = jnp.sum(jnp.where(kiota == i, w_ref[...], 0.0))
    acc_ref[...] += wi * row_ref[0]

    @pl.when(i == K - 1)
    def _():
        ws = ws_ref[...]
        pred = (jnp.sum(ws[0:1] * q_ref[...])
                + jnp.sum(ws[1:2] * acc_ref[...]) + bs_ref[0, 0])
        pred_ref[...] = pred[None, None]
        gi = gi_ref[...] + bih_ref[...]
        gh = gh_ref[...] + bhh_ref[...]
        h = h_ref[...]
        r = jax.nn.sigmoid(gi[0:16] + gh[0:16])
        z = jax.nn.sigmoid(gi[16:32] + gh[16:32])
        n = jnp.tanh(gi[32:48] + r * gh[32:48])
        hnew_ref[...] = (1.0 - z) * n + z * h


def kernel(question, score, questions, hs, Ws, bs, W_ih, W_hh, b_ih, b_hh):
    f32 = jnp.float32
    q2 = question.reshape(QUES, 1)
    h_prev = hs[T - 1, 0]
    h2 = h_prev.reshape(SEQH, 1)
    sel = (score[0] < 0.5).astype(jnp.int32).reshape(1)

    grid_spec = pltpu.PrefetchScalarGridSpec(
        num_scalar_prefetch=1,
        grid=(G1,),
        in_specs=[
            pl.BlockSpec((QUES, 1), lambda i, s: (0, 0)),
            pl.BlockSpec((SEQH, 1), lambda i, s: (0, 0)),
            pl.BlockSpec((QROWS, QUES), lambda i, s: (i, 0)),
            pl.BlockSpec((WROWS, QUES), lambda i, s: (i, s[0])),
            pl.BlockSpec((WROWS, SEQH), lambda i, s: (i, 0)),
        ],
        out_specs=[
            pl.BlockSpec((QROWS, 1), lambda i, s: (i, 0)),
            pl.BlockSpec((WROWS, 1), lambda i, s: (i, 0)),
            pl.BlockSpec((WROWS, 1), lambda i, s: (i, 0)),
        ],
    )
    alpha, gi, gh = pl.pallas_call(
        _matvec_body,
        grid_spec=grid_spec,
        out_shape=[
            jax.ShapeDtypeStruct((T, 1), f32),
            jax.ShapeDtypeStruct((3 * SEQH, 1), f32),
            jax.ShapeDtypeStruct((3 * SEQH, 1), f32),
        ],
    )(sel, q2, h2, questions, W_ih, W_hh)

    idx, w = pl.pallas_call(
        _topk_body,
        out_shape=[
            jax.ShapeDtypeStruct((1, K), jnp.int32),
            jax.ShapeDtypeStruct((1, K), f32),
        ],
    )(alpha.reshape(T // 128, 128))

    pred, h_new = pl.pallas_call(
        _final_body,
        grid_spec=pltpu.PrefetchScalarGridSpec(
            num_scalar_prefetch=1,
            grid=(K,),
            in_specs=[
                pl.BlockSpec((1, K), lambda i, s: (0, 0)),
                pl.BlockSpec((1, 1, SEQH), lambda i, s: (s[i], 0, 0)),
                pl.BlockSpec((1, QUES), lambda i, s: (0, 0)),
                pl.BlockSpec((2, QUES), lambda i, s: (0, 0)),
                pl.BlockSpec((1, 1), lambda i, s: (0, 0)),
                pl.BlockSpec((48, 128), lambda i, s: (0, 0)),
                pl.BlockSpec((48, 128), lambda i, s: (0, 0)),
                pl.BlockSpec((16, 128), lambda i, s: (0, 0)),
                pl.BlockSpec((48, 128), lambda i, s: (0, 0)),
                pl.BlockSpec((48, 128), lambda i, s: (0, 0)),
            ],
            out_specs=[
                pl.BlockSpec((1, 1), lambda i, s: (0, 0)),
                pl.BlockSpec((16, 128), lambda i, s: (0, 0)),
            ],
            scratch_shapes=[
                pltpu.VMEM((1, SEQH), f32),
            ],
        ),
        out_shape=[
            jax.ShapeDtypeStruct((1, 1), f32),
            jax.ShapeDtypeStruct((16, 128), f32),
        ],
    )(
        idx.reshape(K), w, hs,
        question.reshape(1, QUES), Ws.reshape(2, QUES), bs.reshape(1, 1),
        gi.reshape(48, 128), gh.reshape(48, 128), h_prev.reshape(16, 128),
        b_ih.reshape(48, 128), b_hh.reshape(48, 128),
    )
    return (pred.reshape(1), h_new.reshape(1, 1, SEQH))


# speedup vs baseline: 1.9988x; 1.0739x over previous
import functools

import jax
import jax.numpy as jnp
from jax import lax
from jax.experimental import pallas as pl
from jax.experimental.pallas import tpu as pltpu

QUES = 2048
SEQH = 2048
T = 8192
K = 32

G1 = 16
QROWS = T // G1
WROWS = (3 * SEQH) // G1


def _matvec_body(sel_ref, q_ref, h_ref, ques_ref, wih_ref, whh_ref,
                 alpha_ref, gi_ref, gh_ref):
    q = q_ref[...]
    h = h_ref[...]
    alpha_ref[...] = jnp.dot(ques_ref[...], q,
                             preferred_element_type=jnp.float32)
    gi_ref[...] = jnp.dot(wih_ref[...], q,
                          preferred_element_type=jnp.float32)
    gh_ref[...] = jnp.dot(whh_ref[...], h,
                          preferred_element_type=jnp.float32)


def _topk_body(alpha_ref, idx_ref, w_ref):
    a = alpha_ref[...]
    iota = (lax.broadcasted_iota(jnp.int32, (64, 128), 0) * 128
            + lax.broadcasted_iota(jnp.int32, (64, 128), 1))
    kiota = lax.broadcasted_iota(jnp.int32, (1, K), 1)
    neg = jnp.float32(-jnp.inf)

    def step(j, carry):
        a, idxs, vals = carry
        m = jnp.max(a)
        idx = jnp.min(jnp.where(a == m, iota, T))
        idxs = jnp.where(kiota == j, idx, idxs)
        vals = jnp.where(kiota == j, m, vals)
        a = jnp.where(iota == idx, neg, a)
        return a, idxs, vals

    idxs0 = jnp.zeros((1, K), jnp.int32)
    vals0 = jnp.full((1, K), neg, jnp.float32)
    _, idxs, vals = lax.fori_loop(0, K, step, (a, idxs0, vals0))
    e = jnp.exp(vals - jnp.max(vals))
    w = e / jnp.sum(e)
    idx_ref[...] = idxs
    w_ref[...] = w


def _final_body(idx_ref, w_ref, row_ref, q_ref, ws_ref, bs_ref,
                gi_ref, gh_ref, h_ref, bih_ref, bhh_ref,
                pred_ref, hnew_ref, acc_ref):
    i = pl.program_id(0)

    @pl.when(i == 0)
    def _():
        acc_ref[...] = jnp.zeros_like(acc_ref)

    kiota = lax.broadcasted_iota(jnp.int32, (1, K), 1)
    wi = jnp.sum(jnp.where(kiota == i, w_ref[...], 0.0))
    acc_ref[...] += wi * row_ref[0]

    @pl.when(i == K - 1)
    def _():
        ws = ws_ref[...]
        pred = (jnp.sum(ws[0:1] * q_ref[...])
                + jnp.sum(ws[1:2] * acc_ref[...]) + bs_ref[0, 0])
        pred_ref[...] = pred[None, None]
        gi = gi_ref[...] + bih_ref[...]
        gh = gh_ref[...] + bhh_ref[...]
        h = h_ref[...]
        r = jax.nn.sigmoid(gi[0:16] + gh[0:16])
        z = jax.nn.sigmoid(gi[16:32] + gh[16:32])
        n = jnp.tanh(gi[32:48] + r * gh[32:48])
        hnew_ref[...] = (1.0 - z) * n + z * h


def kernel(question, score, questions, hs, Ws, bs, W_ih, W_hh, b_ih, b_hh):
    f32 = jnp.float32
    q2 = question.reshape(QUES, 1)
    h_prev = hs[T - 1, 0]
    h2 = h_prev.reshape(SEQH, 1)
    sel = (score[0] < 0.5).astype(jnp.int32).reshape(1)

    grid_spec = pltpu.PrefetchScalarGridSpec(
        num_scalar_prefetch=1,
        grid=(G1,),
        in_specs=[
            pl.BlockSpec((QUES, 1), lambda i, s: (0, 0)),
            pl.BlockSpec((SEQH, 1), lambda i, s: (0, 0)),
            pl.BlockSpec((QROWS, QUES), lambda i, s: (i, 0)),
            pl.BlockSpec((WROWS, QUES), lambda i, s: (i, s[0])),
            pl.BlockSpec((WROWS, SEQH), lambda i, s: (i, 0)),
        ],
        out_specs=[
            pl.BlockSpec((QROWS, 1), lambda i, s: (i, 0)),
            pl.BlockSpec((WROWS, 1), lambda i, s: (i, 0)),
            pl.BlockSpec((WROWS, 1), lambda i, s: (i, 0)),
        ],
    )
    alpha, gi, gh = pl.pallas_call(
        _matvec_body,
        grid_spec=grid_spec,
        out_shape=[
            jax.ShapeDtypeStruct((T, 1), f32),
            jax.ShapeDtypeStruct((3 * SEQH, 1), f32),
            jax.ShapeDtypeStruct((3 * SEQH, 1), f32),
        ],
    )(sel, q2, h2, questions, W_ih, W_hh)

    idx, w = pl.pallas_call(
        _topk_body,
        out_shape=[
            jax.ShapeDtypeStruct((1, K), jnp.int32),
            jax.ShapeDtypeStruct((1, K), f32),
        ],
    )(alpha.reshape(T // 128, 128))

    pred, h_new = pl.pallas_call(
        _final_body,
        grid_spec=pltpu.PrefetchScalarGridSpec(
            num_scalar_prefetch=1,
            grid=(K,),
            in_specs=[
                pl.BlockSpec((1, K), lambda i, s: (0, 0)),
                pl.BlockSpec((1, 1, SEQH), lambda i, s: (s[i], 0, 0)),
                pl.BlockSpec((1, QUES), lambda i, s: (0, 0)),
                pl.BlockSpec((2, QUES), lambda i, s: (0, 0)),
                pl.BlockSpec((1, 1), lambda i, s: (0, 0)),
                pl.BlockSpec((48, 128), lambda i, s: (0, 0)),
                pl.BlockSpec((48, 128), lambda i, s: (0, 0)),
                pl.BlockSpec((16, 128), lambda i, s: (0, 0)),
                pl.BlockSpec((48, 128), lambda i, s: (0, 0)),
                pl.BlockSpec((48, 128), lambda i, s: (0, 0)),
            ],
            out_specs=[
                pl.BlockSpec((1, 1), lambda i, s: (0, 0)),
                pl.BlockSpec((16, 128), lambda i, s: (0, 0)),
            ],
            scratch_shapes=[
                pltpu.VMEM((1, SEQH), f32),
            ],
        ),
        out_shape=[
            jax.ShapeDtypeStruct((1, 1), f32),
            jax.ShapeDtypeStruct((16, 128), f32),
        ],
    )(
        idx.reshape(K), w, hs,
        question.reshape(1, QUES), Ws.reshape(2, QUES), bs.reshape(1, 1),
        gi.reshape(48, 128), gh.reshape(48, 128), h_prev.reshape(16, 128),
        b_ih.reshape(48, 128), b_hh.reshape(48, 128),
    )
    return (pred.reshape(1), h_new.reshape(1, 1, SEQH))
